# baseline (speedup 1.0000x reference)
import functools
import math

import jax
import jax.numpy as jnp
from jax import lax
from jax.experimental import pallas as pl
from jax.experimental.pallas import tpu as pltpu
from jax.experimental.pallas import tpu_sc as plsc

LANES = 16
NUM_WORKERS = 32


def _positional_encoding(seq, d_model):
    position = jnp.arange(0, seq, dtype=jnp.float32)[:, None]
    div_term = 10000.0 ** (jnp.arange(0, d_model, 2, dtype=jnp.float32) / d_model)
    args = position / div_term
    pe = jnp.zeros((seq, d_model), dtype=jnp.float32)
    pe = pe.at[:, 0::2].set(jnp.sin(args))
    pe = pe.at[:, 1::2].set(jnp.cos(args))
    return pe


@functools.cache
def _build_transpose_call(vocab, d_model):
    nblk = vocab // 128
    tail = vocab - nblk * 128
    rounds = (nblk + NUM_WORKERS - 1) // NUM_WORKERS
    mesh = plsc.VectorSubcoreMesh(core_axis_name="c", subcore_axis_name="s")

    @functools.partial(
        pl.kernel,
        mesh=mesh,
        compiler_params=pltpu.CompilerParams(use_tc_tiling_on_sc=True,
                                             needs_layout_passes=False),
        out_type=jax.ShapeDtypeStruct((vocab // 16, 8, 128), jnp.float32),
        scratch_types=[
            pltpu.VMEM((2, d_model, 128), jnp.float32),
            pltpu.VMEM((2, 8, 8, 128), jnp.float32),
            [pltpu.SemaphoreType.DMA] * 2,
            [pltpu.SemaphoreType.DMA] * 2,
        ],
    )
    def tr_kernel(tt_hbm, tail_hbm, out_hbm, in_v, row_v, sem_i, sem_o):
        wid = lax.axis_index("s") * 2 + lax.axis_index("c")
        lane = lax.iota(jnp.int32, LANES)

        def fire_in(b, blk):
            v0 = blk * 128
            for fb in range(d_model // 8):
                pltpu.async_copy(tt_hbm.at[pl.ds(fb * 8, 8), pl.ds(v0, 128)],
                                 in_v.at[b, pl.ds(fb * 8, 8)], sem_i[b])

        def wait_in(b):
            for fb in range(d_model // 8):
                pltpu.make_async_copy(
                    tt_hbm.at[pl.ds(0, 8), pl.ds(0, 128)],
                    in_v.at[b, pl.ds(fb * 8, 8)], sem_i[b]).wait()

        def fire_out(b, blk):
            pltpu.async_copy(row_v.at[b], out_hbm.at[pl.ds(blk * 8, 8)],
                             sem_o[b])

        def wait_out(b):
            pltpu.make_async_copy(row_v.at[b], out_hbm.at[pl.ds(0, 8)],
                                  sem_o[b]).wait()

        def transpose_block(b, ntok):
            for g in range(ntok // 16):
                for r in range(8):
                    for l0 in range(0, 128, LANES):
                        t = g * 16 + r * 2 + l0 // 64
                        row = lane + (l0 % 64)
                        col = jnp.full((LANES,), t, jnp.int32)
                        vals = plsc.load_gather(in_v.at[b], [row, col])
                        row_v[b, g, r, pl.ds(l0, LANES)] = vals

        blk0 = wid * rounds
        nloc = jnp.minimum(rounds, nblk - blk0)

        fire_in(0, blk0)
        fire_in(1, blk0 + 1)

        def body(i, carry):
            for b in range(2):
                k = i * 2 + b

                @pl.when(k < nloc)
                def _():
                    wait_in(b)

                    @pl.when(k >= 2)
                    def _():
                        wait_out(b)
                    transpose_block(b, 128)
                    fire_out(b, blk0 + k)

                @pl.when(k + 2 < nloc)
                def _():
                    fire_in(b, blk0 + k + 2)
            return carry

        lax.fori_loop(0, (rounds + 1) // 2, body, 0)

        for b in range(2):
            wait_out(b)

        @pl.when(wid == NUM_WORKERS - 1)
        def _():
            pltpu.sync_copy(tail_hbm, row_v.at[0, pl.ds(0, tail // 16)])
            pltpu.sync_copy(row_v.at[0, pl.ds(0, tail // 16)],
                            out_hbm.at[pl.ds(nblk * 8, tail // 16)])

    return tr_kernel


@functools.cache
def _build_sc_call(batch, seq, vocab, d_model):
    bblk = batch // 128
    stile = seq // 8
    fblk = d_model // 8
    scale = math.sqrt(d_model)
    mesh = plsc.VectorSubcoreMesh(core_axis_name="c", subcore_axis_name="s")

    @functools.partial(
        pl.kernel,
        mesh=mesh,
        compiler_params=pltpu.CompilerParams(use_tc_tiling_on_sc=False,
                                             needs_layout_passes=False),
        out_type=jax.ShapeDtypeStruct((seq, fblk, bblk, 8, 128), jnp.float32),
        scratch_types=[
            pltpu.VMEM((stile, 8, 128), jnp.int32),
            pltpu.VMEM((2, 128, d_model), jnp.float32),
            pltpu.VMEM((2, d_model, 128), jnp.float32),
            pltpu.VMEM((seq, d_model), jnp.float32),
            [pltpu.SemaphoreType.DMA] * 2,
            [pltpu.SemaphoreType.DMA] * 2,
        ],
    )
    def sc_kernel(idx_hbm, table_hbm, pe_hbm, out_hbm, idx_v, g_v, o_v, pe_v,
                  sem_g, sem_o):
        wid = lax.axis_index("s") * 2 + lax.axis_index("c")
        pltpu.sync_copy(pe_hbm, pe_v)
        for a in range(stile):
            pltpu.sync_copy(idx_hbm.at[a, wid], idx_v.at[a])

        def fire_gather(b, s):
            a, r = s // 8, s % 8
            pltpu.async_copy(table_hbm.at[idx_v.at[a, r]], g_v.at[b], sem_g[b])

        def wait_gather(b):
            pltpu.make_async_copy(table_hbm.at[idx_v.at[0, 0]], g_v.at[b],
                                  sem_g[b]).wait()

        def fire_out(b, s):
            for fb in range(fblk):
                pltpu.async_copy(o_v.at[b, pl.ds(fb * 8, 8)],
                                 out_hbm.at[s, fb, wid], sem_o[b])

        def wait_out(b):
            for fb in range(fblk):
                pltpu.make_async_copy(o_v.at[b, pl.ds(fb * 8, 8)],
                                      out_hbm.at[0, fb, wid], sem_o[b]).wait()

        lane = lax.iota(jnp.int32, LANES)

        fire_gather(0, 0)
        fire_gather(1, 1)

        def body(s2, carry):
            for b in range(2):
                s = s2 * 2 + b
                wait_gather(b)

                @pl.when(s >= 2)
                def _():
                    wait_out(b)

                for d in range(d_model):
                    if d % LANES == 0:
                        pe16 = pe_v[s, pl.ds(d, LANES)]
                    pe_sd = pe16[d % LANES]
                    col = jnp.full((LANES,), d, jnp.int32)
                    for bl in range(8):
                        row = lane + bl * LANES
                        vals = plsc.load_gather(g_v.at[b], [row, col])
                        o_v[b, d, pl.ds(bl * LANES, LANES)] = (
                            vals * scale + pe_sd)
                fire_out(b, s)

                @pl.when(s + 2 < seq)
                def _():
                    fire_gather(b, s + 2)
            return carry

        lax.fori_loop(0, seq // 2, body, 0)
        wait_out(0)
        wait_out(1)

    return sc_kernel


def kernel(idxs, emb_table):
    batch, seq = idxs.shape
    vocab, d_model = emb_table.shape
    idx4 = (idxs.astype(jnp.int32).T
            .reshape(seq // 8, 8, batch // 128, 128)
            .transpose(0, 2, 1, 3))
    nblk = vocab // 128
    tail_rows = emb_table[nblk * 128:, :].reshape(-1, 8, 128)
    conv = _build_transpose_call(vocab, d_model)(emb_table.T, tail_rows)
    t64 = conv.reshape(vocab, d_model)
    pe = _positional_encoding(seq, d_model)
    out5 = _build_sc_call(batch, seq, vocab, d_model)(idx4, t64, pe)
    return (out5.transpose(2, 4, 0, 1, 3)
            .reshape(batch, seq, d_model))


# speedup vs baseline: 1.0970x; 1.0970x over previous
import functools
import math

import jax
import jax.numpy as jnp
from jax import lax
from jax.experimental import pallas as pl
from jax.experimental.pallas import tpu as pltpu
from jax.experimental.pallas import tpu_sc as plsc

LANES = 16
NUM_WORKERS = 32


def _positional_encoding(seq, d_model):
    position = jnp.arange(0, seq, dtype=jnp.float32)[:, None]
    div_term = 10000.0 ** (jnp.arange(0, d_model, 2, dtype=jnp.float32) / d_model)
    args = position / div_term
    pe = jnp.zeros((seq, d_model), dtype=jnp.float32)
    pe = pe.at[:, 0::2].set(jnp.sin(args))
    pe = pe.at[:, 1::2].set(jnp.cos(args))
    return pe


@functools.cache
def _build_transpose_call(vocab, d_model):
    nblk = vocab // 128
    tail = vocab - nblk * 128
    rounds = (nblk + NUM_WORKERS - 1) // NUM_WORKERS
    mesh = plsc.VectorSubcoreMesh(core_axis_name="c", subcore_axis_name="s")

    @functools.partial(
        pl.kernel,
        mesh=mesh,
        compiler_params=pltpu.CompilerParams(use_tc_tiling_on_sc=True,
                                             needs_layout_passes=False),
        out_type=jax.ShapeDtypeStruct((vocab // 16, 8, 128), jnp.float32),
        scratch_types=[
            pltpu.VMEM((2, d_model, 128), jnp.float32),
            pltpu.VMEM((2, 8, 8, 128), jnp.float32),
            [pltpu.SemaphoreType.DMA] * 2,
            [pltpu.SemaphoreType.DMA] * 2,
        ],
    )
    def tr_kernel(tt_hbm, tail_hbm, out_hbm, in_v, row_v, sem_i, sem_o):
        wid = lax.axis_index("s") * 2 + lax.axis_index("c")
        lane = lax.iota(jnp.int32, LANES)

        def fire_in(b, blk):
            v0 = blk * 128
            for fb in range(d_model // 8):
                pltpu.async_copy(tt_hbm.at[pl.ds(fb * 8, 8), pl.ds(v0, 128)],
                                 in_v.at[b, pl.ds(fb * 8, 8)], sem_i[b])

        def wait_in(b):
            for fb in range(d_model // 8):
                pltpu.make_async_copy(
                    tt_hbm.at[pl.ds(0, 8), pl.ds(0, 128)],
                    in_v.at[b, pl.ds(fb * 8, 8)], sem_i[b]).wait()

        def fire_out(b, blk):
            pltpu.async_copy(row_v.at[b], out_hbm.at[pl.ds(blk * 8, 8)],
                             sem_o[b])

        def wait_out(b):
            pltpu.make_async_copy(row_v.at[b], out_hbm.at[pl.ds(0, 8)],
                                  sem_o[b]).wait()

        def transpose_block(b, ntok):
            def tr_body(gr, _):
                g, r = gr // 8, gr % 8
                t0 = g * 16 + r * 2
                c0 = jnp.broadcast_to(t0, (LANES,)).astype(jnp.int32)
                c1 = jnp.broadcast_to(t0 + 1, (LANES,)).astype(jnp.int32)
                for l0 in range(0, 128, LANES):
                    row = lane + (l0 % 64)
                    col = c0 if l0 < 64 else c1
                    vals = plsc.load_gather(in_v.at[b], [row, col])
                    row_v[b, g, r, pl.ds(l0, LANES)] = vals
                return 0

            lax.fori_loop(0, (ntok // 16) * 8, tr_body, 0)

        blk0 = wid * rounds
        nloc = jnp.minimum(rounds, nblk - blk0)

        fire_in(0, blk0)
        fire_in(1, blk0 + 1)

        def body(i, carry):
            for b in range(2):
                k = i * 2 + b

                @pl.when(k < nloc)
                def _():
                    wait_in(b)

                    @pl.when(k >= 2)
                    def _():
                        wait_out(b)
                    transpose_block(b, 128)
                    fire_out(b, blk0 + k)

                @pl.when(k + 2 < nloc)
                def _():
                    fire_in(b, blk0 + k + 2)
            return carry

        lax.fori_loop(0, (rounds + 1) // 2, body, 0)

        for b in range(2):
            wait_out(b)

        @pl.when(wid == NUM_WORKERS - 1)
        def _():
            pltpu.sync_copy(tail_hbm, row_v.at[0, pl.ds(0, tail // 16)])
            pltpu.sync_copy(row_v.at[0, pl.ds(0, tail // 16)],
                            out_hbm.at[pl.ds(nblk * 8, tail // 16)])

    return tr_kernel


@functools.cache
def _build_sc_call(batch, seq, vocab, d_model):
    bblk = batch // 128
    stile = seq // 8
    fblk = d_model // 8
    scale = math.sqrt(d_model)
    mesh = plsc.VectorSubcoreMesh(core_axis_name="c", subcore_axis_name="s")

    @functools.partial(
        pl.kernel,
        mesh=mesh,
        compiler_params=pltpu.CompilerParams(use_tc_tiling_on_sc=False,
                                             needs_layout_passes=False),
        out_type=jax.ShapeDtypeStruct((seq, fblk, bblk, 8, 128), jnp.float32),
        scratch_types=[
            pltpu.VMEM((stile, 8, 128), jnp.int32),
            pltpu.VMEM((2, 128, d_model), jnp.float32),
            pltpu.VMEM((2, d_model, 128), jnp.float32),
            pltpu.VMEM((seq, d_model), jnp.float32),
            [pltpu.SemaphoreType.DMA] * 2,
            [pltpu.SemaphoreType.DMA] * 2,
        ],
    )
    def sc_kernel(idx_hbm, table_hbm, pe_hbm, out_hbm, idx_v, g_v, o_v, pe_v,
                  sem_g, sem_o):
        wid = lax.axis_index("s") * 2 + lax.axis_index("c")
        pltpu.sync_copy(pe_hbm, pe_v)
        for a in range(stile):
            pltpu.sync_copy(idx_hbm.at[a, wid], idx_v.at[a])

        def fire_gather(b, s):
            a, r = s // 8, s % 8
            pltpu.async_copy(table_hbm.at[idx_v.at[a, r]], g_v.at[b], sem_g[b])

        def wait_gather(b):
            pltpu.make_async_copy(table_hbm.at[idx_v.at[0, 0]], g_v.at[b],
                                  sem_g[b]).wait()

        def fire_out(b, s):
            for fb in range(fblk):
                pltpu.async_copy(o_v.at[b, pl.ds(fb * 8, 8)],
                                 out_hbm.at[s, fb, wid], sem_o[b])

        def wait_out(b):
            for fb in range(fblk):
                pltpu.make_async_copy(o_v.at[b, pl.ds(fb * 8, 8)],
                                      out_hbm.at[0, fb, wid], sem_o[b]).wait()

        lane = lax.iota(jnp.int32, LANES)

        fire_gather(0, 0)
        fire_gather(1, 1)

        def body(s2, carry):
            for b in range(2):
                s = s2 * 2 + b
                wait_gather(b)

                @pl.when(s >= 2)
                def _():
                    wait_out(b)

                srow = jnp.broadcast_to(s, (LANES,)).astype(jnp.int32)

                def sweep(d, _):
                    col = jnp.broadcast_to(d, (LANES,)).astype(jnp.int32)
                    pe16 = plsc.load_gather(pe_v, [srow, col])
                    for bl in range(8):
                        row = lane + bl * LANES
                        vals = plsc.load_gather(g_v.at[b], [row, col])
                        o_v[b, d, pl.ds(bl * LANES, LANES)] = (
                            vals * scale + pe16)
                    return 0

                lax.fori_loop(0, d_model, sweep, 0)
                fire_out(b, s)

                @pl.when(s + 2 < seq)
                def _():
                    fire_gather(b, s + 2)
            return carry

        lax.fori_loop(0, seq // 2, body, 0)
        wait_out(0)
        wait_out(1)

    return sc_kernel


def kernel(idxs, emb_table):
    batch, seq = idxs.shape
    vocab, d_model = emb_table.shape
    idx4 = (idxs.astype(jnp.int32).T
            .reshape(seq // 8, 8, batch // 128, 128)
            .transpose(0, 2, 1, 3))
    nblk = vocab // 128
    tail_rows = emb_table[nblk * 128:, :].reshape(-1, 8, 128)
    conv = _build_transpose_call(vocab, d_model)(emb_table.T, tail_rows)
    t64 = conv.reshape(vocab, d_model)
    pe = _positional_encoding(seq, d_model)
    out5 = _build_sc_call(batch, seq, vocab, d_model)(idx4, t64, pe)
    return (out5.transpose(2, 4, 0, 1, 3)
            .reshape(batch, seq, d_model))


# speedup vs baseline: 2.5772x; 2.3493x over previous
import functools
import math

import jax
import jax.numpy as jnp
from jax import lax
from jax.experimental import pallas as pl
from jax.experimental.pallas import tpu as pltpu
from jax.experimental.pallas import tpu_sc as plsc

D_MODEL = 64
SEQ = 200
CHUNK = 200
IDX_MINOR = 100
GATHERS = CHUNK // IDX_MINOR
LANES = 16
NUM_WORKERS = 32
NBUF = 4
LOOKAHEAD = 2


def _positional_encoding(seq, d_model):
    position = jnp.arange(0, seq, dtype=jnp.float32)[:, None]
    div_term = 10000.0 ** (jnp.arange(0, d_model, 2, dtype=jnp.float32) / d_model)
    args = position / div_term
    pe = jnp.zeros((seq, d_model), dtype=jnp.float32)
    pe = pe.at[:, 0::2].set(jnp.sin(args))
    pe = pe.at[:, 1::2].set(jnp.cos(args))
    return pe


@functools.cache
def _build_sc_call(rows, d_model):
    rows_per_worker = rows // NUM_WORKERS
    chunks_per_worker = rows_per_worker // CHUNK
    slab_streams = rows_per_worker // IDX_MINOR
    scale = math.sqrt(d_model)
    mesh = plsc.VectorSubcoreMesh(core_axis_name="c", subcore_axis_name="s")

    @functools.partial(
        pl.kernel,
        mesh=mesh,
        compiler_params=pltpu.CompilerParams(use_tc_tiling_on_sc=False),
        out_type=jax.ShapeDtypeStruct((rows, d_model), jnp.float32),
        scratch_types=[
            pltpu.VMEM((slab_streams, IDX_MINOR), jnp.int32),
            pltpu.VMEM((NBUF, CHUNK, d_model), jnp.float32),
            pltpu.VMEM((CHUNK, d_model), jnp.float32),
            [pltpu.SemaphoreType.DMA] * NBUF,
            [pltpu.SemaphoreType.DMA] * NBUF,
        ],
    )
    def sc_kernel(idx_hbm, table_hbm, pe_hbm, out_hbm, idx_v, dest_v, pe_v,
                  sem_g, sem_o):
        wid = lax.axis_index("s") * 2 + lax.axis_index("c")
        pltpu.sync_copy(pe_hbm, pe_v)
        pltpu.sync_copy(idx_hbm.at[wid], idx_v)

        def fire_gather(b, l):
            for j in range(GATHERS):
                pltpu.async_copy(
                    table_hbm.at[idx_v.at[l * GATHERS + j]],
                    dest_v.at[b, pl.ds(j * IDX_MINOR, IDX_MINOR)],
                    sem_g[b],
                )

        def wait_gather(b):
            for j in range(GATHERS):
                pltpu.make_async_copy(
                    table_hbm.at[idx_v.at[j]],
                    dest_v.at[b, pl.ds(j * IDX_MINOR, IDX_MINOR)],
                    sem_g[b],
                ).wait()

        def fire_out(b, l):
            base = (wid * chunks_per_worker + l) * CHUNK
            pltpu.async_copy(dest_v.at[b], out_hbm.at[pl.ds(base, CHUNK)],
                             sem_o[b])

        def wait_out(b):
            pltpu.make_async_copy(dest_v.at[b],
                                  out_hbm.at[pl.ds(0, CHUNK)], sem_o[b]).wait()

        for b in range(LOOKAHEAD):
            fire_gather(b, b)

        def body(c4, carry):
            for b in range(NBUF):
                l = c4 * NBUF + b
                wait_gather(b)

                def sweep(r, _):
                    for d in range(d_model // LANES):
                        sl = pl.ds(d * LANES, LANES)
                        dest_v[b, r, sl] = dest_v[b, r, sl] * scale + pe_v[r, sl]
                    return 0

                lax.fori_loop(0, CHUNK, sweep, 0)
                fire_out(b, l)
                bf = (b + LOOKAHEAD) % NBUF
                lf = l + LOOKAHEAD

                @pl.when(lf < chunks_per_worker)
                def _():
                    @pl.when(lf >= NBUF)
                    def _():
                        wait_out(bf)
                    fire_gather(bf, lf)
            return carry

        lax.fori_loop(0, chunks_per_worker // NBUF, body, 0)
        for b in range(NBUF):
            wait_out(b)

    return sc_kernel


def kernel(idxs, emb_table):
    batch, seq = idxs.shape
    vocab, d_model = emb_table.shape
    rows = batch * seq
    idx_resh = idxs.astype(jnp.int32).reshape(
        NUM_WORKERS, rows // (NUM_WORKERS * IDX_MINOR), IDX_MINOR)
    pe_tiled = jnp.tile(_positional_encoding(seq, d_model), (CHUNK // seq, 1))
    out_flat = _build_sc_call(rows, d_model)(idx_resh, emb_table, pe_tiled)
    return out_flat.reshape(batch, seq, d_model)
